# baseline (device time: 536486 ns/iter reference)
import jax
import jax.numpy as jnp
from jax import lax
from jax.experimental import pallas as pl
from jax.experimental.pallas import tpu as pltpu

N_CHUNK = 8


def kernel(x):
    m, n = x.shape
    half_n = n // 2
    rows = m // N_CHUNK

    def body(x_ref, out_ref, send_buf, recv_buf, copy_sem,
             fill_sems, drain_sems, send_sems, recv_sems):
        my_x = lax.axis_index("x")
        my_y = lax.axis_index("y")
        other_x = 1 - my_x

        barrier_sem = pltpu.get_barrier_semaphore()
        pl.semaphore_signal(
            barrier_sem, inc=1,
            device_id=(other_x, my_y), device_id_type=pl.DeviceIdType.MESH,
        )
        pl.semaphore_wait(barrier_sem, 1)

        local = pltpu.make_async_copy(
            x_ref.at[:, pl.ds(my_x * half_n, half_n)],
            out_ref.at[pl.ds(my_x * m, m), :],
            copy_sem,
        )
        local.start()

        def make_fill(c, slot):
            return pltpu.make_async_copy(
                x_ref.at[pl.ds(c * rows, rows), pl.ds(other_x * half_n, half_n)],
                send_buf.at[slot],
                fill_sems.at[slot],
            )

        def make_rdma(c, slot):
            return pltpu.make_async_remote_copy(
                src_ref=send_buf.at[slot],
                dst_ref=recv_buf.at[c],
                send_sem=send_sems.at[slot],
                recv_sem=recv_sems.at[c],
                device_id=(other_x, my_y),
                device_id_type=pl.DeviceIdType.MESH,
            )

        def make_drain(c):
            return pltpu.make_async_copy(
                recv_buf.at[c],
                out_ref.at[pl.ds(other_x * m + c * rows, rows), :],
                drain_sems.at[c],
            )

        rdmas = []
        make_fill(0, 0).start()
        for c in range(N_CHUNK):
            slot = c % 2
            make_fill(c, slot).wait()
            rdma = make_rdma(c, slot)
            rdma.start()
            rdmas.append(rdma)
            if c + 1 < N_CHUNK:
                if c >= 1:
                    rdmas[c - 1].wait_send()
                make_fill(c + 1, (c + 1) % 2).start()

        for rdma in rdmas[-2:]:
            rdma.wait_send()
        for c in range(N_CHUNK):
            rdmas[c].wait_recv()
            make_drain(c).start()
        for c in range(N_CHUNK):
            make_drain(c).wait()
        local.wait()

    return pl.pallas_call(
        body,
        out_shape=jax.ShapeDtypeStruct((2 * m, half_n), x.dtype),
        in_specs=[pl.BlockSpec(memory_space=pl.ANY)],
        out_specs=pl.BlockSpec(memory_space=pl.ANY),
        scratch_shapes=[
            pltpu.VMEM((2, rows, half_n), x.dtype),
            pltpu.VMEM((N_CHUNK, rows, half_n), x.dtype),
            pltpu.SemaphoreType.DMA,
            pltpu.SemaphoreType.DMA((2,)),
            pltpu.SemaphoreType.DMA((N_CHUNK,)),
            pltpu.SemaphoreType.DMA((2,)),
            pltpu.SemaphoreType.DMA((N_CHUNK,)),
        ],
        compiler_params=pltpu.CompilerParams(collective_id=0),
    )(x)


# device time: 216940 ns/iter; 2.4730x vs baseline; 2.4730x over previous
import jax
import jax.numpy as jnp
from jax import lax
from jax.experimental import pallas as pl
from jax.experimental.pallas import tpu as pltpu

N_CHUNK = 8
N_SLOT = 3


def kernel(x):
    m, n = x.shape
    half_n = n // 2
    rows = m // N_CHUNK

    def body(x_ref, out_ref, xbuf, rbuf,
             fill_sems, ldrain_sems, rdrain_sems, send_sems, recv_sems):
        my_x = lax.axis_index("x")
        my_y = lax.axis_index("y")
        other_x = 1 - my_x

        barrier_sem = pltpu.get_barrier_semaphore()
        pl.semaphore_signal(
            barrier_sem, inc=1,
            device_id=(other_x, my_y), device_id_type=pl.DeviceIdType.MESH,
        )
        pl.semaphore_wait(barrier_sem, 1)

        def make_fill(c):
            slot = c % N_SLOT
            return pltpu.make_async_copy(
                x_ref.at[pl.ds(c * rows, rows), :],
                xbuf.at[slot],
                fill_sems.at[slot],
            )

        def make_rdma(c):
            slot = c % N_SLOT
            return pltpu.make_async_remote_copy(
                src_ref=xbuf.at[slot, :, pl.ds(other_x * half_n, half_n)],
                dst_ref=rbuf.at[c],
                send_sem=send_sems.at[slot],
                recv_sem=recv_sems.at[c],
                device_id=(other_x, my_y),
                device_id_type=pl.DeviceIdType.MESH,
            )

        def make_ldrain(c):
            slot = c % N_SLOT
            return pltpu.make_async_copy(
                xbuf.at[slot, :, pl.ds(my_x * half_n, half_n)],
                out_ref.at[pl.ds(my_x * m + c * rows, rows), :],
                ldrain_sems.at[c],
            )

        def make_rdrain(c):
            return pltpu.make_async_copy(
                rbuf.at[c],
                out_ref.at[pl.ds(other_x * m + c * rows, rows), :],
                rdrain_sems.at[c],
            )

        for c in range(N_SLOT):
            make_fill(c).start()

        rdmas = []
        for c in range(N_CHUNK):
            make_fill(c).wait()
            rdma = make_rdma(c)
            rdma.start()
            rdmas.append(rdma)
            make_ldrain(c).start()
            if c + N_SLOT < N_CHUNK:
                rdmas[c].wait_send()
                make_ldrain(c).wait()
                make_fill(c + N_SLOT).start()

        for c in range(max(0, N_CHUNK - N_SLOT), N_CHUNK):
            rdmas[c].wait_send()
            make_ldrain(c).wait()

        for c in range(N_CHUNK):
            rdmas[c].wait_recv()
            make_rdrain(c).start()
        for c in range(N_CHUNK):
            make_rdrain(c).wait()

    return pl.pallas_call(
        body,
        out_shape=jax.ShapeDtypeStruct((2 * m, half_n), x.dtype),
        in_specs=[pl.BlockSpec(memory_space=pl.ANY)],
        out_specs=pl.BlockSpec(memory_space=pl.ANY),
        scratch_shapes=[
            pltpu.VMEM((N_SLOT, rows, n), x.dtype),
            pltpu.VMEM((N_CHUNK, rows, half_n), x.dtype),
            pltpu.SemaphoreType.DMA((N_SLOT,)),
            pltpu.SemaphoreType.DMA((N_CHUNK,)),
            pltpu.SemaphoreType.DMA((N_CHUNK,)),
            pltpu.SemaphoreType.DMA((N_SLOT,)),
            pltpu.SemaphoreType.DMA((N_CHUNK,)),
        ],
        compiler_params=pltpu.CompilerParams(collective_id=0),
    )(x)


# device time: 135113 ns/iter; 3.9706x vs baseline; 1.6056x over previous
import jax
import jax.numpy as jnp
from jax import lax
from jax.experimental import pallas as pl
from jax.experimental.pallas import tpu as pltpu

N_CHUNK = 16
N_HALF = N_CHUNK // 2


def kernel(x):
    m, n = x.shape
    half_n = n // 2
    rows = m // N_CHUNK

    def body(x_ref, out_ref, xbuf, rbx, rby, fill_sems, send_sems,
             yfs_sems, recv_x_sems, recv_y_sems,
             ldrain_sems, rdx_sems, rdy_sems):
        my_x = lax.axis_index("x")
        my_y = lax.axis_index("y")
        other_x = 1 - my_x
        other_y = 1 - my_y

        barrier_sem = pltpu.get_barrier_semaphore()
        pl.semaphore_signal(
            barrier_sem, inc=1,
            device_id=(other_x, my_y), device_id_type=pl.DeviceIdType.MESH,
        )
        pl.semaphore_signal(
            barrier_sem, inc=1,
            device_id=(my_x, other_y), device_id_type=pl.DeviceIdType.MESH,
        )
        pl.semaphore_wait(barrier_sem, 2)

        def chunk_row(t):
            if t < N_HALF:
                return my_y * N_HALF + t
            return other_y * N_HALF + (t - N_HALF)

        def make_fill(t):
            return pltpu.make_async_copy(
                x_ref.at[pl.ds(chunk_row(t) * rows, rows), :],
                xbuf.at[t],
                fill_sems.at[t],
            )

        def make_rdma_x(t):
            return pltpu.make_async_remote_copy(
                src_ref=xbuf.at[t, :, pl.ds(other_x * half_n, half_n)],
                dst_ref=rbx.at[t],
                send_sem=send_sems.at[t],
                recv_sem=recv_x_sems.at[t],
                device_id=(other_x, my_y),
                device_id_type=pl.DeviceIdType.MESH,
            )

        def make_rdma_y(j):
            return pltpu.make_async_remote_copy(
                src_ref=rbx.at[j],
                dst_ref=rby.at[j],
                send_sem=yfs_sems.at[j],
                recv_sem=recv_y_sems.at[j],
                device_id=(my_x, other_y),
                device_id_type=pl.DeviceIdType.MESH,
            )

        def make_ldrain(t):
            return pltpu.make_async_copy(
                xbuf.at[t, :, pl.ds(my_x * half_n, half_n)],
                out_ref.at[pl.ds(my_x * m + chunk_row(t) * rows, rows), :],
                ldrain_sems.at[t],
            )

        def make_rdrain_x(j):
            return pltpu.make_async_copy(
                rbx.at[j],
                out_ref.at[
                    pl.ds(other_x * m + (my_y * N_HALF + j) * rows, rows), :
                ],
                rdx_sems.at[j],
            )

        def make_rdrain_y(j):
            return pltpu.make_async_copy(
                rby.at[j],
                out_ref.at[
                    pl.ds(other_x * m + (other_y * N_HALF + j) * rows, rows), :
                ],
                rdy_sems.at[j],
            )

        for t in range(N_CHUNK):
            make_fill(t).start()

        rdmas_x = {}
        for t in range(N_CHUNK):
            make_fill(t).wait()
            if t < N_HALF:
                rdma = make_rdma_x(t)
                rdma.start()
                rdmas_x[t] = rdma
            make_ldrain(t).start()

        yfwds = {}
        for j in range(N_HALF):
            rdmas_x[j].wait_recv()
            fwd = make_rdma_y(j)
            fwd.start()
            yfwds[j] = fwd
            make_rdrain_x(j).start()

        for j in range(N_HALF):
            yfwds[j].wait_recv()
            make_rdrain_y(j).start()

        for j in range(N_HALF):
            rdmas_x[j].wait_send()
            yfwds[j].wait_send()
            make_rdrain_x(j).wait()
            make_rdrain_y(j).wait()
        for t in range(N_CHUNK):
            make_ldrain(t).wait()

    return pl.pallas_call(
        body,
        out_shape=jax.ShapeDtypeStruct((2 * m, half_n), x.dtype),
        in_specs=[pl.BlockSpec(memory_space=pl.ANY)],
        out_specs=pl.BlockSpec(memory_space=pl.ANY),
        scratch_shapes=[
            pltpu.VMEM((N_CHUNK, rows, n), x.dtype),
            pltpu.VMEM((N_HALF, rows, half_n), x.dtype),
            pltpu.VMEM((N_HALF, rows, half_n), x.dtype),
            pltpu.SemaphoreType.DMA((N_CHUNK,)),
            pltpu.SemaphoreType.DMA((N_HALF,)),
            pltpu.SemaphoreType.DMA((N_HALF,)),
            pltpu.SemaphoreType.DMA((N_HALF,)),
            pltpu.SemaphoreType.DMA((N_HALF,)),
            pltpu.SemaphoreType.DMA((N_CHUNK,)),
            pltpu.SemaphoreType.DMA((N_HALF,)),
            pltpu.SemaphoreType.DMA((N_HALF,)),
        ],
        compiler_params=pltpu.CompilerParams(
            collective_id=0, vmem_limit_bytes=56 * 1024 * 1024
        ),
    )(x)


# device time: 133694 ns/iter; 4.0128x vs baseline; 1.0106x over previous
import jax
import jax.numpy as jnp
from jax import lax
from jax.experimental import pallas as pl
from jax.experimental.pallas import tpu as pltpu

N_CHUNK = 16
N_HALF = N_CHUNK // 2


def kernel(x):
    m, n = x.shape
    half_n = n // 2
    rows = m // N_CHUNK

    def body(x_ref, out_ref, xbuf, lbuf, rbx, rby, fill_sems, send_sems,
             yfs_sems, recv_x_sems, recv_y_sems,
             ldrain_sems, rdx_sems, rdy_sems):
        my_x = lax.axis_index("x")
        my_y = lax.axis_index("y")
        other_x = 1 - my_x
        other_y = 1 - my_y

        barrier_sem = pltpu.get_barrier_semaphore()
        pl.semaphore_signal(
            barrier_sem, inc=1,
            device_id=(other_x, my_y), device_id_type=pl.DeviceIdType.MESH,
        )
        pl.semaphore_signal(
            barrier_sem, inc=1,
            device_id=(my_x, other_y), device_id_type=pl.DeviceIdType.MESH,
        )
        pl.semaphore_wait(barrier_sem, 2)

        def chunk_row(t):
            if t < N_HALF:
                return my_y * N_HALF + t
            return other_y * N_HALF + (t - N_HALF)

        def make_fill(t):
            if t < N_HALF:
                return pltpu.make_async_copy(
                    x_ref.at[pl.ds(chunk_row(t) * rows, rows), :],
                    xbuf.at[t],
                    fill_sems.at[t],
                )
            return pltpu.make_async_copy(
                x_ref.at[
                    pl.ds(chunk_row(t) * rows, rows),
                    pl.ds(my_x * half_n, half_n),
                ],
                lbuf.at[t - N_HALF],
                fill_sems.at[t],
            )

        def make_rdma_x(t):
            return pltpu.make_async_remote_copy(
                src_ref=xbuf.at[t, :, pl.ds(other_x * half_n, half_n)],
                dst_ref=rbx.at[t],
                send_sem=send_sems.at[t],
                recv_sem=recv_x_sems.at[t],
                device_id=(other_x, my_y),
                device_id_type=pl.DeviceIdType.MESH,
            )

        def make_rdma_y(j):
            return pltpu.make_async_remote_copy(
                src_ref=rbx.at[j],
                dst_ref=rby.at[j],
                send_sem=yfs_sems.at[j],
                recv_sem=recv_y_sems.at[j],
                device_id=(my_x, other_y),
                device_id_type=pl.DeviceIdType.MESH,
            )

        def make_ldrain(t):
            if t < N_HALF:
                src_ref = xbuf.at[t, :, pl.ds(my_x * half_n, half_n)]
            else:
                src_ref = lbuf.at[t - N_HALF]
            return pltpu.make_async_copy(
                src_ref,
                out_ref.at[pl.ds(my_x * m + chunk_row(t) * rows, rows), :],
                ldrain_sems.at[t],
            )

        def make_rdrain_x(j):
            return pltpu.make_async_copy(
                rbx.at[j],
                out_ref.at[
                    pl.ds(other_x * m + (my_y * N_HALF + j) * rows, rows), :
                ],
                rdx_sems.at[j],
            )

        def make_rdrain_y(j):
            return pltpu.make_async_copy(
                rby.at[j],
                out_ref.at[
                    pl.ds(other_x * m + (other_y * N_HALF + j) * rows, rows), :
                ],
                rdy_sems.at[j],
            )

        for t in range(N_CHUNK):
            make_fill(t).start()

        rdmas_x = {}
        for t in range(N_CHUNK):
            make_fill(t).wait()
            if t < N_HALF:
                rdma = make_rdma_x(t)
                rdma.start()
                rdmas_x[t] = rdma
            make_ldrain(t).start()

        yfwds = {}
        for j in range(N_HALF):
            rdmas_x[j].wait_recv()
            fwd = make_rdma_y(j)
            fwd.start()
            yfwds[j] = fwd
            make_rdrain_x(j).start()

        for j in range(N_HALF):
            yfwds[j].wait_recv()
            make_rdrain_y(j).start()

        for j in range(N_HALF):
            rdmas_x[j].wait_send()
            yfwds[j].wait_send()
            make_rdrain_x(j).wait()
            make_rdrain_y(j).wait()
        for t in range(N_CHUNK):
            make_ldrain(t).wait()

    return pl.pallas_call(
        body,
        out_shape=jax.ShapeDtypeStruct((2 * m, half_n), x.dtype),
        in_specs=[pl.BlockSpec(memory_space=pl.ANY)],
        out_specs=pl.BlockSpec(memory_space=pl.ANY),
        scratch_shapes=[
            pltpu.VMEM((N_HALF, rows, n), x.dtype),
            pltpu.VMEM((N_HALF, rows, half_n), x.dtype),
            pltpu.VMEM((N_HALF, rows, half_n), x.dtype),
            pltpu.VMEM((N_HALF, rows, half_n), x.dtype),
            pltpu.SemaphoreType.DMA((N_CHUNK,)),
            pltpu.SemaphoreType.DMA((N_HALF,)),
            pltpu.SemaphoreType.DMA((N_HALF,)),
            pltpu.SemaphoreType.DMA((N_HALF,)),
            pltpu.SemaphoreType.DMA((N_HALF,)),
            pltpu.SemaphoreType.DMA((N_CHUNK,)),
            pltpu.SemaphoreType.DMA((N_HALF,)),
            pltpu.SemaphoreType.DMA((N_HALF,)),
        ],
        compiler_params=pltpu.CompilerParams(
            collective_id=0, vmem_limit_bytes=56 * 1024 * 1024
        ),
    )(x)
